# Initial kernel scaffold; baseline (speedup 1.0000x reference)
import jax
import jax.numpy as jnp
from jax.experimental import pallas as pl


def kernel(x_paper, x_author, edge_index_cites, edge_index_writes, edge_index_written_by, params):
    raise NotImplementedError("write your pallas kernel here")


# speedup vs baseline: 4.7133x; 4.7133x over previous
import functools

import jax
import jax.numpy as jnp
from jax.experimental import pallas as pl
from jax.experimental.pallas import tpu as pltpu

_N_PAPER = 50000
_N_AUTHOR = 50000
_E = 200000
_HID = 128
_CFGS = [(4, 32), (1, 128)]


def _mm_bias_body(x_ref, w_ref, b_ref, o_ref):
    o_ref[...] = (
        jnp.dot(x_ref[...], w_ref[...], preferred_element_type=jnp.float32)
        + b_ref[...]
    )


def _mm_bias(x, w, b, block=2000):
    n, k = x.shape
    m = w.shape[1]
    return pl.pallas_call(
        _mm_bias_body,
        grid=(n // block,),
        in_specs=[
            pl.BlockSpec((block, k), lambda i: (i, 0)),
            pl.BlockSpec((k, m), lambda i: (0, 0)),
            pl.BlockSpec((1, m), lambda i: (0, 0)),
        ],
        out_specs=pl.BlockSpec((block, m), lambda i: (i, 0)),
        out_shape=jax.ShapeDtypeStruct((n, m), jnp.float32),
    )(x, w, b.reshape(1, m))


def _ln_relu_res_body(x_ref, g_ref, b_ref, r_ref, o_ref):
    x = x_ref[...]
    mu = jnp.mean(x, axis=-1, keepdims=True)
    var = jnp.mean((x - mu) ** 2, axis=-1, keepdims=True)
    z = (x - mu) / jnp.sqrt(var + 1e-5) * g_ref[...] + b_ref[...]
    o_ref[...] = jnp.maximum(z, 0.0) + r_ref[...]


def _ln_relu_res(x, g, b, res, block=2000):
    n, m = x.shape
    return pl.pallas_call(
        _ln_relu_res_body,
        grid=(n // block,),
        in_specs=[
            pl.BlockSpec((block, m), lambda i: (i, 0)),
            pl.BlockSpec((1, m), lambda i: (0, 0)),
            pl.BlockSpec((1, m), lambda i: (0, 0)),
            pl.BlockSpec((block, m), lambda i: (i, 0)),
        ],
        out_specs=pl.BlockSpec((block, m), lambda i: (i, 0)),
        out_shape=jax.ShapeDtypeStruct((n, m), jnp.float32),
    )(x, g.reshape(1, m), b.reshape(1, m), res)


def _edge_phase(hs, es, ed, src, dst, heads, oc, num_dst):
    e = es[src] + ed[dst]
    e = jnp.where(e >= 0, e, 0.2 * e)
    ex = jnp.exp(e)
    den = jax.ops.segment_sum(ex, dst, num_segments=num_dst)
    alpha = ex / (den[dst] + 1e-16)
    msg = hs[src].reshape(-1, heads, oc) * alpha[:, :, None]
    out = jax.ops.segment_sum(msg.reshape(-1, heads * oc), dst,
                              num_segments=num_dst)
    return out


def _blockdiag(a):
    h, oc = a.shape
    m = jnp.zeros((h, oc, h), jnp.float32)
    m = m.at[jnp.arange(h), :, jnp.arange(h)].set(a)
    return m.reshape(h * oc, h)


def kernel(x_paper, x_author, edge_index_cites, edge_index_writes,
           edge_index_written_by, params):
    h = {
        "paper": _mm_bias(x_paper, params["Win_paper"], params["bin_paper"]),
        "author": _mm_bias(x_author, params["Win_author"], params["bin_author"]),
    }
    edges = {
        "cites": ("paper", "paper", edge_index_cites),
        "writes": ("author", "paper", edge_index_writes),
        "written_by": ("paper", "author", edge_index_written_by),
    }
    nnum = {"paper": _N_PAPER, "author": _N_AUTHOR}
    for l, (hd_, oc) in enumerate(_CFGS):
        h_prev = dict(h)
        out = {t: jnp.zeros((nnum[t], hd_ * oc), jnp.float32) for t in nnum}
        for r, (s, d, ei) in edges.items():
            W = params["W%d_%s" % (l, r)]
            hs = _mm_bias(h[s], W, jnp.zeros((hd_ * oc,), jnp.float32))
            asd = _blockdiag(params["as%d_%s" % (l, r)])
            add = _blockdiag(params["ad%d_%s" % (l, r)])
            es = hs @ asd
            if s == d:
                hd_mat = hs
            else:
                hd_mat = _mm_bias(h[d], W, jnp.zeros((hd_ * oc,), jnp.float32))
            ed = hd_mat @ add
            conv = _edge_phase(hs, es, ed, ei[0], ei[1], hd_, oc, nnum[d])
            out[d] = out[d] + conv + params["b%d_%s" % (l, r)]
        for t in nnum:
            h[t] = _ln_relu_res(out[t], params["ln%d_%s_g" % (l, t)],
                                params["ln%d_%s_b" % (l, t)], h_prev[t])
    return h["paper"], h["author"]


# speedup vs baseline: 4.7690x; 1.0118x over previous
import functools

import jax
import jax.numpy as jnp
from jax.experimental import pallas as pl

_N_PAPER = 50000
_N_AUTHOR = 50000
_CFGS = [(4, 32), (1, 128)]


def _mm_bias_body(x_ref, w_ref, b_ref, o_ref):
    o_ref[...] = (
        jnp.dot(x_ref[...], w_ref[...], preferred_element_type=jnp.float32)
        + b_ref[...]
    )


def _mm_bias(x, w, b, block=2000):
    n, k = x.shape
    m = w.shape[1]
    return pl.pallas_call(
        _mm_bias_body,
        grid=(n // block,),
        in_specs=[
            pl.BlockSpec((block, k), lambda i: (i, 0)),
            pl.BlockSpec((k, m), lambda i: (0, 0)),
            pl.BlockSpec((1, m), lambda i: (0, 0)),
        ],
        out_specs=pl.BlockSpec((block, m), lambda i: (i, 0)),
        out_shape=jax.ShapeDtypeStruct((n, m), jnp.float32),
    )(x, w, b.reshape(1, m))


def _ln_relu_res_body(x_ref, g_ref, b_ref, r_ref, o_ref):
    x = x_ref[...]
    mu = jnp.mean(x, axis=-1, keepdims=True)
    var = jnp.mean((x - mu) ** 2, axis=-1, keepdims=True)
    z = (x - mu) / jnp.sqrt(var + 1e-5) * g_ref[...] + b_ref[...]
    o_ref[...] = jnp.maximum(z, 0.0) + r_ref[...]


def _ln_relu_res(x, g, b, res, block=2000):
    n, m = x.shape
    return pl.pallas_call(
        _ln_relu_res_body,
        grid=(n // block,),
        in_specs=[
            pl.BlockSpec((block, m), lambda i: (i, 0)),
            pl.BlockSpec((1, m), lambda i: (0, 0)),
            pl.BlockSpec((1, m), lambda i: (0, 0)),
            pl.BlockSpec((block, m), lambda i: (i, 0)),
        ],
        out_specs=pl.BlockSpec((block, m), lambda i: (i, 0)),
        out_shape=jax.ShapeDtypeStruct((n, m), jnp.float32),
    )(x, g.reshape(1, m), b.reshape(1, m), res)


def _sort_edges(ei):
    order = jnp.argsort(ei[1])
    return ei[0][order], ei[1][order]


def _edge_phase(hs, es, ed, src, dst, heads, oc, num_dst):
    e = es[src] + ed[dst]
    e = jnp.where(e >= 0, e, 0.2 * e)
    ex = jnp.exp(e)
    den = jax.ops.segment_sum(ex, dst, num_segments=num_dst,
                              indices_are_sorted=True)
    alpha = ex / (den[dst] + 1e-16)
    msg = hs[src].reshape(-1, heads, oc) * alpha[:, :, None]
    return jax.ops.segment_sum(msg.reshape(-1, heads * oc), dst,
                               num_segments=num_dst,
                               indices_are_sorted=True)


def _blockdiag(a):
    h, oc = a.shape
    m = jnp.zeros((h, oc, h), jnp.float32)
    m = m.at[jnp.arange(h), :, jnp.arange(h)].set(a)
    return m.reshape(h * oc, h)


def kernel(x_paper, x_author, edge_index_cites, edge_index_writes,
           edge_index_written_by, params):
    h = {
        "paper": _mm_bias(x_paper, params["Win_paper"], params["bin_paper"]),
        "author": _mm_bias(x_author, params["Win_author"], params["bin_author"]),
    }
    edges = {
        "cites": ("paper", "paper", _sort_edges(edge_index_cites)),
        "writes": ("author", "paper", _sort_edges(edge_index_writes)),
        "written_by": ("paper", "author", _sort_edges(edge_index_written_by)),
    }
    nnum = {"paper": _N_PAPER, "author": _N_AUTHOR}
    for l, (hd_, oc) in enumerate(_CFGS):
        h_prev = dict(h)
        out = {t: jnp.zeros((nnum[t], hd_ * oc), jnp.float32) for t in nnum}
        for r, (s, d, ei) in edges.items():
            W = params["W%d_%s" % (l, r)]
            hs = _mm_bias(h[s], W, jnp.zeros((hd_ * oc,), jnp.float32))
            asd = _blockdiag(params["as%d_%s" % (l, r)])
            add = _blockdiag(params["ad%d_%s" % (l, r)])
            es = hs @ asd
            if s == d:
                hd_mat = hs
            else:
                hd_mat = _mm_bias(h[d], W, jnp.zeros((hd_ * oc,), jnp.float32))
            ed = hd_mat @ add
            conv = _edge_phase(hs, es, ed, ei[0], ei[1], hd_, oc,
                               nnum[d])
            out[d] = out[d] + conv + params["b%d_%s" % (l, r)]
        for t in nnum:
            h[t] = _ln_relu_res(out[t], params["ln%d_%s_g" % (l, t)],
                                params["ln%d_%s_b" % (l, t)], h_prev[t])
    return h["paper"], h["author"]


# speedup vs baseline: 9.7481x; 2.0440x over previous
import functools

import jax
import jax.numpy as jnp
from jax import lax
from jax.experimental import pallas as pl
from jax.experimental.pallas import tpu as pltpu
from jax.experimental.pallas import tpu_sc as plsc

_N_PAPER = 50000
_N_AUTHOR = 50000
_E = 200000
_CFGS = [(4, 32), (1, 128)]

_NC = 2
_NS = 16
_B = 128
_E_PAD = 212992
_T_NB = _E_PAD // (_NS * _B)
_N_DST = 50000
_DEN_R = 50048
_ZDR = _DEN_R // _NS


def _mm_bias_body(x_ref, w_ref, b_ref, o_ref):
    o_ref[...] = (
        jnp.dot(x_ref[...], w_ref[...], preferred_element_type=jnp.float32)
        + b_ref[...]
    )


def _mm_bias(x, w, b, block=2000):
    n, k = x.shape
    m = w.shape[1]
    return pl.pallas_call(
        _mm_bias_body,
        grid=(n // block,),
        in_specs=[
            pl.BlockSpec((block, k), lambda i: (i, 0)),
            pl.BlockSpec((k, m), lambda i: (0, 0)),
            pl.BlockSpec((1, m), lambda i: (0, 0)),
        ],
        out_specs=pl.BlockSpec((block, m), lambda i: (i, 0)),
        out_shape=jax.ShapeDtypeStruct((n, m), jnp.float32),
    )(x, w, b.reshape(1, m))


def _ln_relu_res_body(x_ref, g_ref, b_ref, r_ref, o_ref):
    x = x_ref[...]
    mu = jnp.mean(x, axis=-1, keepdims=True)
    var = jnp.mean((x - mu) ** 2, axis=-1, keepdims=True)
    z = (x - mu) / jnp.sqrt(var + 1e-5) * g_ref[...] + b_ref[...]
    o_ref[...] = jnp.maximum(z, 0.0) + r_ref[...]


def _ln_relu_res(x, g, b, res, block=2000):
    n, m = x.shape
    return pl.pallas_call(
        _ln_relu_res_body,
        grid=(n // block,),
        in_specs=[
            pl.BlockSpec((block, m), lambda i: (i, 0)),
            pl.BlockSpec((1, m), lambda i: (0, 0)),
            pl.BlockSpec((1, m), lambda i: (0, 0)),
            pl.BlockSpec((block, m), lambda i: (i, 0)),
        ],
        out_specs=pl.BlockSpec((block, m), lambda i: (i, 0)),
        out_shape=jax.ShapeDtypeStruct((n, m), jnp.float32),
    )(x, g.reshape(1, m), b.reshape(1, m), res)


def _make_sc_alpha():
    mesh = plsc.VectorSubcoreMesh(core_axis_name="c", subcore_axis_name="s")

    @functools.partial(
        pl.kernel,
        out_type=jax.ShapeDtypeStruct((_E_PAD, 16), jnp.float32),
        mesh=mesh,
        compiler_params=pltpu.CompilerParams(use_tc_tiling_on_sc=False),
        scratch_types=[
            pltpu.VMEM((_T_NB, _B), jnp.int32),
            pltpu.VMEM((_T_NB, _B), jnp.int32),
            pltpu.VMEM((_B, 16), jnp.float32),
            pltpu.VMEM((_B, 16), jnp.float32),
            pltpu.VMEM((_B, 16), jnp.float32),
            pltpu.VMEM((_B, 16), jnp.float32),
            pltpu.VMEM((_B, 16), jnp.float32),
            pltpu.VMEM((_B, 16), jnp.float32),
            pltpu.VMEM_SHARED((_DEN_R, 16), jnp.float32),
        ],
    )
    def k(src_hbm, dst_hbm, es_hbm, ed_hbm, alpha_out,
          src_res, dst_res, es_rows, ed_rows, ex_rows, den_rows, zden,
          alpha_g, den_sh):
        c = lax.axis_index("c")
        t = lax.axis_index("s")
        zf = jnp.zeros((16,), jnp.float32)

        pltpu.sync_copy(src_hbm.at[pl.ds(t * _T_NB, _T_NB)], src_res)
        pltpu.sync_copy(dst_hbm.at[pl.ds(t * _T_NB, _T_NB)], dst_res)

        def zero_zden(j, carry):
            zden[j, pl.ds(0, 16)] = zf
            return carry
        lax.fori_loop(0, _B, zero_zden, 0)

        dbase = t * _ZDR
        nfull_d = _ZDR // _B

        def zero_den(q, carry):
            pltpu.sync_copy(zden, den_sh.at[pl.ds(dbase + q * _B, _B)])
            return carry
        lax.fori_loop(0, nfull_d, zero_den, 0)
        rem_d = _ZDR - nfull_d * _B
        pltpu.sync_copy(zden.at[pl.ds(0, rem_d)],
                        den_sh.at[pl.ds(dbase + nfull_d * _B, rem_d)])
        plsc.subcore_barrier()

        def s1(blk, carry):
            sidx = src_res.at[blk]
            didx = dst_res.at[blk]
            pltpu.sync_copy(es_hbm.at[sidx], es_rows)
            pltpu.sync_copy(ed_hbm.at[didx], ed_rows)

            def row1(j, c2):
                a = es_rows[j, pl.ds(0, 16)]
                b_ = ed_rows[j, pl.ds(0, 16)]
                e = a + b_
                e = jnp.where(e >= 0, e, 0.2 * e)
                ex_rows[j, pl.ds(0, 16)] = jnp.exp(e)
                return c2
            lax.fori_loop(0, _B, row1, 0)
            pltpu.sync_copy(ex_rows, den_sh.at[didx], add=True)
            return carry
        lax.fori_loop(0, _T_NB, s1, 0)
        plsc.subcore_barrier()

        half = _T_NB // _NC

        def s2(i, carry):
            blk = c * half + i
            sidx = src_res.at[blk]
            didx = dst_res.at[blk]
            pltpu.sync_copy(es_hbm.at[sidx], es_rows)
            pltpu.sync_copy(ed_hbm.at[didx], ed_rows)
            pltpu.sync_copy(den_sh.at[didx], den_rows)

            def row2(j, c2):
                a = es_rows[j, pl.ds(0, 16)]
                b_ = ed_rows[j, pl.ds(0, 16)]
                e = a + b_
                e = jnp.where(e >= 0, e, 0.2 * e)
                dnv = den_rows[j, pl.ds(0, 16)]
                alpha_g[j, pl.ds(0, 16)] = jnp.exp(e) / (dnv + 1e-16)
                return c2
            lax.fori_loop(0, _B, row2, 0)
            pltpu.sync_copy(
                alpha_g,
                alpha_out.at[pl.ds((t * _T_NB + blk) * _B, _B)])
            return carry
        lax.fori_loop(0, half, s2, 0)

    return k


_SC_ALPHA = _make_sc_alpha()


def _pad_sort_edges(ei):
    src = jnp.pad(ei[0], (0, _E_PAD - _E))
    dst = jnp.pad(ei[1], (0, _E_PAD - _E), constant_values=_N_DST)
    order = jnp.argsort(dst)
    return src[order], dst[order]


def _msg_phase(hs, alpha, src, dst, heads, oc, num_dst):
    msg = hs[src].reshape(-1, heads, oc) * alpha[:, :heads, None]
    return jax.ops.segment_sum(msg.reshape(-1, heads * oc), dst,
                               num_segments=num_dst,
                               indices_are_sorted=True)


def _blockdiag(a):
    h, oc = a.shape
    m = jnp.zeros((h, oc, 16), jnp.float32)
    m = m.at[jnp.arange(h), :, jnp.arange(h)].set(a)
    return m.reshape(h * oc, 16)


def kernel(x_paper, x_author, edge_index_cites, edge_index_writes,
           edge_index_written_by, params):
    h = {
        "paper": _mm_bias(x_paper, params["Win_paper"], params["bin_paper"]),
        "author": _mm_bias(x_author, params["Win_author"], params["bin_author"]),
    }
    edges = {
        "cites": ("paper", "paper", _pad_sort_edges(edge_index_cites)),
        "writes": ("author", "paper", _pad_sort_edges(edge_index_writes)),
        "written_by": ("paper", "author", _pad_sort_edges(edge_index_written_by)),
    }
    nnum = {"paper": _N_PAPER, "author": _N_AUTHOR}
    for l, (hd_, oc) in enumerate(_CFGS):
        h_prev = dict(h)
        out = {t: jnp.zeros((nnum[t], hd_ * oc), jnp.float32) for t in nnum}
        for r, (s, d, ei) in edges.items():
            W = params["W%d_%s" % (l, r)]
            hs = _mm_bias(h[s], W, jnp.zeros((hd_ * oc,), jnp.float32))
            asd = _blockdiag(params["as%d_%s" % (l, r)])
            add = _blockdiag(params["ad%d_%s" % (l, r)])
            es = hs @ asd
            if s == d:
                hd_mat = hs
            else:
                hd_mat = _mm_bias(h[d], W, jnp.zeros((hd_ * oc,), jnp.float32))
            ed = jnp.pad(hd_mat @ add, ((0, 8), (0, 0)))
            src2d = ei[0].reshape(_NS * _T_NB, _B)
            dst2d = ei[1].reshape(_NS * _T_NB, _B)
            alpha = _SC_ALPHA(src2d, dst2d, es, ed)
            conv = _msg_phase(hs, alpha, ei[0], ei[1], hd_, oc,
                              nnum[d])
            out[d] = out[d] + conv + params["b%d_%s" % (l, r)]
        for t in nnum:
            h[t] = _ln_relu_res(out[t], params["ln%d_%s_g" % (l, t)],
                                params["ln%d_%s_b" % (l, t)], h_prev[t])
    return h["paper"], h["author"]


# speedup vs baseline: 9.7505x; 1.0002x over previous
import functools

import jax
import jax.numpy as jnp
from jax import lax
from jax.experimental import pallas as pl
from jax.experimental.pallas import tpu as pltpu
from jax.experimental.pallas import tpu_sc as plsc

_N_PAPER = 50000
_N_AUTHOR = 50000
_E = 200000
_CFGS = [(4, 32), (1, 128)]

_NC = 2
_NS = 16
_B = 128
_E_PAD = 212992
_T_NB = _E_PAD // (_NS * _B)
_N_DST = 50000
_DEN_R = 50048
_ZDR = _DEN_R // _NS


def _mm_bias_body(x_ref, w_ref, b_ref, o_ref):
    o_ref[...] = (
        jnp.dot(x_ref[...], w_ref[...], preferred_element_type=jnp.float32)
        + b_ref[...]
    )


def _mm_bias(x, w, b, block=2000):
    n, k = x.shape
    m = w.shape[1]
    return pl.pallas_call(
        _mm_bias_body,
        grid=(n // block,),
        in_specs=[
            pl.BlockSpec((block, k), lambda i: (i, 0)),
            pl.BlockSpec((k, m), lambda i: (0, 0)),
            pl.BlockSpec((1, m), lambda i: (0, 0)),
        ],
        out_specs=pl.BlockSpec((block, m), lambda i: (i, 0)),
        out_shape=jax.ShapeDtypeStruct((n, m), jnp.float32),
    )(x, w, b.reshape(1, m))


def _ln_relu_res_body(x_ref, g_ref, b_ref, r_ref, o_ref):
    x = x_ref[...]
    mu = jnp.mean(x, axis=-1, keepdims=True)
    var = jnp.mean((x - mu) ** 2, axis=-1, keepdims=True)
    z = (x - mu) / jnp.sqrt(var + 1e-5) * g_ref[...] + b_ref[...]
    o_ref[...] = jnp.maximum(z, 0.0) + r_ref[...]


def _ln_relu_res(x, g, b, res, block=2000):
    n, m = x.shape
    return pl.pallas_call(
        _ln_relu_res_body,
        grid=(n // block,),
        in_specs=[
            pl.BlockSpec((block, m), lambda i: (i, 0)),
            pl.BlockSpec((1, m), lambda i: (0, 0)),
            pl.BlockSpec((1, m), lambda i: (0, 0)),
            pl.BlockSpec((block, m), lambda i: (i, 0)),
        ],
        out_specs=pl.BlockSpec((block, m), lambda i: (i, 0)),
        out_shape=jax.ShapeDtypeStruct((n, m), jnp.float32),
    )(x, g.reshape(1, m), b.reshape(1, m), res)


def _make_sc_alpha():
    mesh = plsc.VectorSubcoreMesh(core_axis_name="c", subcore_axis_name="s")

    @functools.partial(
        pl.kernel,
        out_type=jax.ShapeDtypeStruct((_E_PAD, 16), jnp.float32),
        mesh=mesh,
        compiler_params=pltpu.CompilerParams(use_tc_tiling_on_sc=False),
        scratch_types=[
            pltpu.VMEM((_T_NB, _B), jnp.int32),
            pltpu.VMEM((_T_NB, _B), jnp.int32),
            pltpu.VMEM((_B, 16), jnp.float32),
            pltpu.VMEM((_B, 16), jnp.float32),
            pltpu.VMEM((_B, 16), jnp.float32),
            pltpu.VMEM((_B, 16), jnp.float32),
            pltpu.VMEM((_B, 16), jnp.float32),
            pltpu.VMEM((_B, 16), jnp.float32),
            pltpu.VMEM_SHARED((_DEN_R, 16), jnp.float32),
        ],
    )
    def k(src_hbm, dst_hbm, es_hbm, ed_hbm, alpha_out,
          src_res, dst_res, es_rows, ed_rows, ex_rows, den_rows, zden,
          alpha_g, den_sh):
        c = lax.axis_index("c")
        t = lax.axis_index("s")
        zf = jnp.zeros((16,), jnp.float32)

        pltpu.sync_copy(src_hbm.at[pl.ds(t * _T_NB, _T_NB)], src_res)
        pltpu.sync_copy(dst_hbm.at[pl.ds(t * _T_NB, _T_NB)], dst_res)

        def zero_zden(j, carry):
            zden[j, pl.ds(0, 16)] = zf
            return carry
        lax.fori_loop(0, _B, zero_zden, 0)

        dbase = t * _ZDR
        nfull_d = _ZDR // _B

        def zero_den(q, carry):
            pltpu.sync_copy(zden, den_sh.at[pl.ds(dbase + q * _B, _B)])
            return carry
        lax.fori_loop(0, nfull_d, zero_den, 0)
        rem_d = _ZDR - nfull_d * _B
        pltpu.sync_copy(zden.at[pl.ds(0, rem_d)],
                        den_sh.at[pl.ds(dbase + nfull_d * _B, rem_d)])
        plsc.subcore_barrier()

        def s1(blk, carry):
            sidx = src_res.at[blk]
            didx = dst_res.at[blk]
            pltpu.sync_copy(es_hbm.at[sidx], es_rows)
            pltpu.sync_copy(ed_hbm.at[didx], ed_rows)

            def row1(j, c2):
                a = es_rows[j, pl.ds(0, 16)]
                b_ = ed_rows[j, pl.ds(0, 16)]
                e = a + b_
                e = jnp.where(e >= 0, e, 0.2 * e)
                ex_rows[j, pl.ds(0, 16)] = jnp.exp(e)
                return c2
            lax.fori_loop(0, _B, row1, 0)
            pltpu.sync_copy(ex_rows, den_sh.at[didx], add=True)
            return carry
        lax.fori_loop(0, _T_NB, s1, 0)
        plsc.subcore_barrier()

        half = _T_NB // _NC

        def s2(i, carry):
            blk = c * half + i
            sidx = src_res.at[blk]
            didx = dst_res.at[blk]
            pltpu.sync_copy(es_hbm.at[sidx], es_rows)
            pltpu.sync_copy(ed_hbm.at[didx], ed_rows)
            pltpu.sync_copy(den_sh.at[didx], den_rows)

            def row2(j, c2):
                a = es_rows[j, pl.ds(0, 16)]
                b_ = ed_rows[j, pl.ds(0, 16)]
                e = a + b_
                e = jnp.where(e >= 0, e, 0.2 * e)
                dnv = den_rows[j, pl.ds(0, 16)]
                alpha_g[j, pl.ds(0, 16)] = jnp.exp(e) / (dnv + 1e-16)
                return c2
            lax.fori_loop(0, _B, row2, 0)
            pltpu.sync_copy(
                alpha_g,
                alpha_out.at[pl.ds((t * _T_NB + blk) * _B, _B)])
            return carry
        lax.fori_loop(0, half, s2, 0)

    return k


_SC_ALPHA_CACHE = []


def _sc_alpha(*args):
    if not _SC_ALPHA_CACHE:
        _SC_ALPHA_CACHE.append(_make_sc_alpha())
    return _SC_ALPHA_CACHE[0](*args)


def _pad_sort_edges(ei):
    src = jnp.pad(ei[0], (0, _E_PAD - _E))
    dst = jnp.pad(ei[1], (0, _E_PAD - _E), constant_values=_N_DST)
    order = jnp.argsort(dst)
    return src[order], dst[order]


def _msg_phase(hs, alpha, src, dst, heads, oc, num_dst):
    msg = hs[src].reshape(-1, heads, oc) * alpha[:, :heads, None]
    return jax.ops.segment_sum(msg.reshape(-1, heads * oc), dst,
                               num_segments=num_dst,
                               indices_are_sorted=True)


def _blockdiag(a):
    h, oc = a.shape
    m = jnp.zeros((h, oc, 16), jnp.float32)
    m = m.at[jnp.arange(h), :, jnp.arange(h)].set(a)
    return m.reshape(h * oc, 16)


def kernel(x_paper, x_author, edge_index_cites, edge_index_writes,
           edge_index_written_by, params):
    h = {
        "paper": _mm_bias(x_paper, params["Win_paper"], params["bin_paper"]),
        "author": _mm_bias(x_author, params["Win_author"], params["bin_author"]),
    }
    edges = {
        "cites": ("paper", "paper", _pad_sort_edges(edge_index_cites)),
        "writes": ("author", "paper", _pad_sort_edges(edge_index_writes)),
        "written_by": ("paper", "author", _pad_sort_edges(edge_index_written_by)),
    }
    nnum = {"paper": _N_PAPER, "author": _N_AUTHOR}
    for l, (hd_, oc) in enumerate(_CFGS):
        h_prev = dict(h)
        out = {t: jnp.zeros((nnum[t], hd_ * oc), jnp.float32) for t in nnum}
        for r, (s, d, ei) in edges.items():
            W = params["W%d_%s" % (l, r)]
            hs = _mm_bias(h[s], W, jnp.zeros((hd_ * oc,), jnp.float32))
            asd = _blockdiag(params["as%d_%s" % (l, r)])
            add = _blockdiag(params["ad%d_%s" % (l, r)])
            es = hs @ asd
            if s == d:
                hd_mat = hs
            else:
                hd_mat = _mm_bias(h[d], W, jnp.zeros((hd_ * oc,), jnp.float32))
            ed = jnp.pad(hd_mat @ add, ((0, 8), (0, 0)))
            src2d = ei[0].reshape(_NS * _T_NB, _B)
            dst2d = ei[1].reshape(_NS * _T_NB, _B)
            alpha = _sc_alpha(src2d, dst2d, es, ed)
            conv = _msg_phase(hs, alpha, ei[0], ei[1], hd_, oc,
                              nnum[d])
            out[d] = out[d] + conv + params["b%d_%s" % (l, r)]
        for t in nnum:
            h[t] = _ln_relu_res(out[t], params["ln%d_%s_g" % (l, t)],
                                params["ln%d_%s_b" % (l, t)], h_prev[t])
    return h["paper"], h["author"]
